# Initial kernel scaffold; baseline (speedup 1.0000x reference)
import jax
import jax.numpy as jnp
from jax.experimental import pallas as pl


def kernel(x, edge_index, batch, W0, b0, W1, b1, W2, b2, lin1_w, lin1_b, lin2_w, lin2_b):
    raise NotImplementedError("write your pallas kernel here")


# speedup vs baseline: 2.1776x; 2.1776x over previous
import functools

import jax
import jax.numpy as jnp
from jax import lax
from jax.experimental import pallas as pl
from jax.experimental.pallas import tpu as pltpu
from jax.experimental.pallas import tpu_sc as plsc

N = 10000
NP = 10240
E = 320000
D = 128
H = 256
C = 2
G = 64

NC = 2
NS = 16
K = 128
NCH = E // K
CH_PER_TILE = -(-NCH // NS)
ROWS_PT = NP // NS

RB = 256
NRB = NP // RB


HALF = NP // 2
ROWS_Q = HALF // NS


@functools.lru_cache(maxsize=None)
def _make_sc_agg():
  mesh = plsc.VectorSubcoreMesh(core_axis_name="c", subcore_axis_name="s",
                                num_cores=NC, num_subcores=NS)

  @functools.partial(
      pl.kernel,
      mesh=mesh,
      out_type=jax.ShapeDtypeStruct((NC, NP, 128), jnp.float32),
      scratch_types=[
          pltpu.VMEM((K,), jnp.int32),
          pltpu.VMEM((K,), jnp.int32),
          pltpu.VMEM((K, 128), jnp.float32),
          pltpu.VMEM((ROWS_Q, 128), jnp.float32),
          pltpu.VMEM_SHARED((HALF, 128), jnp.float32),
          pltpu.SemaphoreType.DMA,
      ],
  )
  def agg(table_hbm, src2_hbm, dst_hbm, zeros_hbm, out_hbm,
          sidx, didx, rows, stage, acc, sem):
    c = lax.axis_index("c")
    s = lax.axis_index("s")
    r0 = s * ROWS_Q

    for p in (0, 1):
      lo = p * HALF

      pltpu.sync_copy(zeros_hbm.at[pl.ds(r0, ROWS_Q), :], stage)
      pltpu.sync_copy(stage, acc.at[pl.ds(r0, ROWS_Q), :])
      plsc.subcore_barrier()

      def body(i, carry):
        ch = i * NS + s

        @pl.when(ch < NCH)
        def _():
          e0 = ch * K
          pltpu.sync_copy(src2_hbm.at[pl.ds(c * E + e0, K)], sidx)
          pltpu.sync_copy(dst_hbm.at[pl.ds(e0, K)], didx)
          for j in range(K // 16):
            sl = pl.ds(j * 16, 16)
            d = didx[sl]
            keep = (d >= lo) & (d < lo + HALF)
            sidx[sl] = jnp.where(keep, sidx[sl], -1)
            didx[sl] = jnp.where(keep, d - lo, -1)
          pltpu.async_copy(
              table_hbm.at[plsc.Indices(sidx, ignored_value=-1)], rows, sem
          ).wait()
          pltpu.sync_copy(
              rows, acc.at[plsc.Indices(didx, ignored_value=-1)], add=True
          )

        return carry

      lax.fori_loop(0, CH_PER_TILE, body, 0)
      plsc.subcore_barrier()

      pltpu.sync_copy(acc.at[pl.ds(r0, ROWS_Q), :], stage)
      pltpu.sync_copy(stage, out_hbm.at[c, pl.ds(lo + r0, ROWS_Q), :])
      plsc.subcore_barrier()

  return agg


def _matmul0_body(x_ref, w_ref, o_ref):
  y = jnp.dot(x_ref[...], w_ref[...], preferred_element_type=jnp.float32)
  o_ref[0] = y[:, :128]
  o_ref[1] = y[:, 128:]


_tc_matmul0 = pl.pallas_call(
    _matmul0_body,
    grid=(NRB,),
    in_specs=[
        pl.BlockSpec((RB, D), lambda i: (i, 0)),
        pl.BlockSpec((D, H), lambda i: (0, 0)),
    ],
    out_specs=pl.BlockSpec((NC, RB, 128), lambda i: (0, i, 0)),
    out_shape=jax.ShapeDtypeStruct((NC, NP, 128), jnp.float32),
)


def _update_body(z_ref, a_ref, b_ref, w_ref, o_ref):
  u0 = z_ref[0] + a_ref[0] + b_ref[:, :128]
  u1 = z_ref[1] + a_ref[1] + b_ref[:, 128:]
  ss = (jnp.sum(u0 * u0, axis=1, keepdims=True)
        + jnp.sum(u1 * u1, axis=1, keepdims=True))
  d = jnp.maximum(jnp.sqrt(ss), 1e-12)
  t0 = jnp.maximum(u0 / d, 0.0)
  t1 = jnp.maximum(u1 / d, 0.0)
  y = (jnp.dot(t0, w_ref[:128, :], preferred_element_type=jnp.float32)
       + jnp.dot(t1, w_ref[128:, :], preferred_element_type=jnp.float32))
  o_ref[0] = y[:, :128]
  o_ref[1] = y[:, 128:]


_tc_update = pl.pallas_call(
    _update_body,
    grid=(NRB,),
    in_specs=[
        pl.BlockSpec((NC, RB, 128), lambda i: (0, i, 0)),
        pl.BlockSpec((NC, RB, 128), lambda i: (0, i, 0)),
        pl.BlockSpec((1, H), lambda i: (0, 0)),
        pl.BlockSpec((H, H), lambda i: (0, 0)),
    ],
    out_specs=pl.BlockSpec((NC, RB, 128), lambda i: (0, i, 0)),
    out_shape=jax.ShapeDtypeStruct((NC, NP, 128), jnp.float32),
)


def _pool_head_body(z_ref, a_ref, b_ref, batch_ref, w1_ref, b1_ref,
                    w2_ref, b2_ref, o_ref, acc_ref):
  i = pl.program_id(0)

  @pl.when(i == 0)
  def _():
    acc_ref[...] = jnp.zeros_like(acc_ref)

  u0 = z_ref[0] + a_ref[0] + b_ref[:, :128]
  u1 = z_ref[1] + a_ref[1] + b_ref[:, 128:]
  ss = (jnp.sum(u0 * u0, axis=1, keepdims=True)
        + jnp.sum(u1 * u1, axis=1, keepdims=True))
  d = jnp.maximum(jnp.sqrt(ss), 1e-12)
  t0 = jnp.maximum(u0 / d, 0.0)
  t1 = jnp.maximum(u1 / d, 0.0)

  ids = batch_ref[0, 0, :]
  gi = lax.broadcasted_iota(jnp.int32, (G, RB), 0)
  onehot = (gi == ids[None, :]).astype(jnp.float32)
  acc_ref[:, :128] += jnp.dot(onehot, t0, preferred_element_type=jnp.float32)
  acc_ref[:, 128:] += jnp.dot(onehot, t1, preferred_element_type=jnp.float32)

  @pl.when(i == NRB - 1)
  def _():
    z = jnp.dot(acc_ref[...], w1_ref[...],
                preferred_element_type=jnp.float32) + b1_ref[...]
    z = jnp.maximum(z, 0.0)
    o_ref[...] = jnp.dot(z, w2_ref[...],
                         preferred_element_type=jnp.float32) + b2_ref[...]


_tc_pool_head = pl.pallas_call(
    _pool_head_body,
    grid=(NRB,),
    in_specs=[
        pl.BlockSpec((NC, RB, 128), lambda i: (0, i, 0)),
        pl.BlockSpec((NC, RB, 128), lambda i: (0, i, 0)),
        pl.BlockSpec((1, H), lambda i: (0, 0)),
        pl.BlockSpec((1, 1, RB), lambda i: (i, 0, 0)),
        pl.BlockSpec((H, H), lambda i: (0, 0)),
        pl.BlockSpec((1, H), lambda i: (0, 0)),
        pl.BlockSpec((H, 128), lambda i: (0, 0)),
        pl.BlockSpec((1, 128), lambda i: (0, 0)),
    ],
    out_specs=pl.BlockSpec((G, 128), lambda i: (0, 0)),
    out_shape=jax.ShapeDtypeStruct((G, 128), jnp.float32),
    scratch_shapes=[pltpu.VMEM((G, H), jnp.float32)],
)


def kernel(x, edge_index, batch, W0, b0, W1, b1, W2, b2,
           lin1_w, lin1_b, lin2_w, lin2_b):
  src = edge_index[0]
  dst = edge_index[1]
  src2 = jnp.concatenate([src, src + NP])
  zeros = jnp.zeros((HALF, 128), jnp.float32)
  sc_agg = _make_sc_agg()

  xpad = jnp.pad(x, ((0, NP - N), (0, 0)))
  z1 = _tc_matmul0(xpad, W0)
  a1 = sc_agg(z1.reshape(2 * NP, 128), src2, dst, zeros)
  z2 = _tc_update(z1, a1, b0.reshape(1, H), W1)
  a2 = sc_agg(z2.reshape(2 * NP, 128), src2, dst, zeros)
  z3 = _tc_update(z2, a2, b1.reshape(1, H), W2)
  a3 = sc_agg(z3.reshape(2 * NP, 128), src2, dst, zeros)

  batch_pad = jnp.pad(batch, (0, NP - N), constant_values=G)
  logits_pad = _tc_pool_head(
      z3, a3, b2.reshape(1, H), batch_pad.reshape(NRB, 1, RB),
      lin1_w, lin1_b.reshape(1, H),
      jnp.pad(lin2_w, ((0, 0), (0, 128 - C))),
      jnp.pad(lin2_b, (0, 128 - C)).reshape(1, 128),
  )
  return logits_pad[:, :C]


# speedup vs baseline: 7.0704x; 3.2468x over previous
import functools

import jax
import jax.numpy as jnp
from jax import lax
from jax.experimental import pallas as pl
from jax.experimental.pallas import tpu as pltpu
from jax.experimental.pallas import tpu_sc as plsc

N = 10000
NP = 10240
E = 320000
D = 128
H = 256
C = 2
G = 64

NC = 2
NS = 16
K = 128
NCH = E // K
CH_PER_TILE = -(-NCH // NS)
ROWS_PT = NP // NS

RB = 256
NRB = NP // RB


NSEG = 2
SEG = NP // NSEG
ROWS_Q = SEG // NS
SR = 80
NSR = ROWS_Q // SR


FL = (NCH // NS) // 4 * 4
REM = NCH - FL * NS
FL4 = FL // 4


@functools.lru_cache(maxsize=None)
def _make_sc_agg():
  mesh = plsc.VectorSubcoreMesh(core_axis_name="c", subcore_axis_name="s",
                                num_cores=NC, num_subcores=NS)

  @functools.partial(
      pl.kernel,
      mesh=mesh,
      out_type=jax.ShapeDtypeStruct((NC, NP, 128), jnp.float32),
      scratch_types=[
          pltpu.VMEM((2, K), jnp.int32),
          pltpu.VMEM((2, K), jnp.int32),
          pltpu.VMEM((K,), jnp.int32),
          pltpu.VMEM((K,), jnp.int32),
          pltpu.VMEM((K,), jnp.int32),
          pltpu.VMEM((K,), jnp.int32),
          pltpu.VMEM((K,), jnp.int32),
          pltpu.VMEM((K,), jnp.int32),
          pltpu.VMEM((K, 128), jnp.float32),
          pltpu.VMEM((K, 128), jnp.float32),
          pltpu.VMEM((K, 128), jnp.float32),
          pltpu.VMEM((K, 128), jnp.float32),
          pltpu.VMEM((SR, 128), jnp.float32),
          pltpu.VMEM_SHARED((SEG, 128), jnp.float32),
          pltpu.SemaphoreType.DMA,
          pltpu.SemaphoreType.DMA,
          pltpu.SemaphoreType.DMA,
          pltpu.SemaphoreType.DMA,
          pltpu.SemaphoreType.DMA,
          pltpu.SemaphoreType.DMA,
      ],
  )
  def agg(table_hbm, pk_hbm, zeros_hbm, out_hbm,
          pb0, pb1, mi0, mi1, md00, md01, md10, md11,
          r00, r01, r10, r11, stage, acc,
          semi0, semi1, semg0, semg1, sems0, sems1):
    c = lax.axis_index("c")
    s = lax.axis_index("s")
    pb = [pb0, pb1]
    mi = [mi0, mi1]
    md = [[md00, md01], [md10, md11]]
    rows = [[r00, r01], [r10, r11]]
    semi = [semi0, semi1]
    semg = [semg0, semg1]
    sems = [sems0, sems1]
    r0 = s * ROWS_Q

    def load(kk, b):
      ch = jnp.minimum(kk * NS + s, NCH - 1)
      pltpu.async_copy(pk_hbm.at[c, ch], pb[b], semi[b])

    def wait_load(b):
      pltpu.make_async_copy(pk_hbm.at[c, 0], pb[b], semi[b]).wait()

    def gather(b, ph):
      pltpu.async_copy(
          table_hbm.at[plsc.Indices(mi[b], ignored_value=-1)],
          rows[b][ph], semg[b])

    def wait_gather(b, ph):
      pltpu.make_async_copy(
          table_hbm.at[plsc.Indices(mi[b], ignored_value=-1)],
          rows[b][ph], semg[b]).wait()

    def scatter(b, ph):
      pltpu.async_copy(
          rows[b][ph], acc.at[plsc.Indices(md[b][ph], ignored_value=-1)],
          sems[b], add=True)

    def wait_scatter(b, ph):
      pltpu.make_async_copy(
          rows[b][ph], acc.at[plsc.Indices(md[b][ph], ignored_value=-1)],
          sems[b]).wait()

    for p in range(NSEG):
      lo = p * SEG

      with jax.named_scope(f"zero{p}"):
        pltpu.sync_copy(zeros_hbm.at[pl.ds(0, SR), :], stage)
        for q in range(NSR):
          pltpu.sync_copy(stage, acc.at[pl.ds(r0 + q * SR, SR), :])
        plsc.subcore_barrier()

      def mask(b, ph):
        for j in range(K // 16):
          sl = pl.ds(j * 16, 16)
          sv = pb[b][0, sl]
          d = pb[b][1, sl]
          if p == 0:
            keep = d < SEG
          elif p == NSEG - 1:
            keep = d >= lo
          else:
            keep = (d >= lo) & (d < lo + SEG)
          dl = d - lo
          mi[b][sl] = jnp.where(keep, sv, -1)
          md[b][ph][sl] = jnp.where(keep, dl, -1)

      scope_e = jax.named_scope(f"edges{p}")
      scope_e.__enter__()
      load(0, 0)
      load(1, 1)
      wait_load(0); mask(0, 0); load(2, 0); gather(0, 0)
      wait_load(1); mask(1, 0); load(3, 1); gather(1, 0)
      wait_load(0); wait_gather(0, 0); scatter(0, 0)
      mask(0, 1); load(4, 0); gather(0, 1)
      wait_load(1); wait_gather(1, 0); scatter(1, 0)
      mask(1, 1); load(5, 1); gather(1, 1)

      def lbody(u, carry):
        base = u * 4
        for off, (b, ph) in enumerate(((0, 0), (1, 0), (0, 1), (1, 1))):
          kk = base + off
          wait_scatter(b, ph)
          wait_gather(b, 1 - ph)
          scatter(b, 1 - ph)
          wait_load(b)
          mask(b, ph)
          load(kk + 2, b)
          gather(b, ph)
        return carry

      lax.fori_loop(1, FL4, lbody, 0)

      wait_gather(0, 1); scatter(0, 1)
      wait_gather(1, 1); scatter(1, 1)
      wait_scatter(0, 0)
      wait_load(0)

      @pl.when(s < REM)
      def _():
        mask(0, 0)
        gather(0, 0)
        wait_gather(0, 0)
        scatter(0, 0)
        wait_scatter(0, 0)

      wait_load(1)
      wait_scatter(1, 0)
      wait_scatter(0, 1)
      wait_scatter(1, 1)
      plsc.subcore_barrier()
      scope_e.__exit__(None, None, None)

      with jax.named_scope(f"copyout{p}"):
        for q in range(NSR):
          pltpu.sync_copy(acc.at[pl.ds(r0 + q * SR, SR), :], stage)
          pltpu.sync_copy(stage, out_hbm.at[c, pl.ds(lo + r0 + q * SR, SR), :])
        plsc.subcore_barrier()

  return agg


def _matmul0_body(x_ref, w_ref, o_ref):
  y = jnp.dot(x_ref[...], w_ref[...], preferred_element_type=jnp.float32)
  o_ref[0] = y[:, :128]
  o_ref[1] = y[:, 128:]


_tc_matmul0 = pl.pallas_call(
    _matmul0_body,
    grid=(NRB,),
    in_specs=[
        pl.BlockSpec((RB, D), lambda i: (i, 0)),
        pl.BlockSpec((D, H), lambda i: (0, 0)),
    ],
    out_specs=pl.BlockSpec((NC, RB, 128), lambda i: (0, i, 0)),
    out_shape=jax.ShapeDtypeStruct((NC, NP, 128), jnp.float32),
)


def _update_body(z_ref, a_ref, b_ref, w_ref, o_ref):
  u0 = z_ref[0] + a_ref[0] + b_ref[:, :128]
  u1 = z_ref[1] + a_ref[1] + b_ref[:, 128:]
  ss = (jnp.sum(u0 * u0, axis=1, keepdims=True)
        + jnp.sum(u1 * u1, axis=1, keepdims=True))
  d = jnp.maximum(jnp.sqrt(ss), 1e-12)
  t0 = jnp.maximum(u0 / d, 0.0)
  t1 = jnp.maximum(u1 / d, 0.0)
  y = (jnp.dot(t0, w_ref[:128, :], preferred_element_type=jnp.float32)
       + jnp.dot(t1, w_ref[128:, :], preferred_element_type=jnp.float32))
  o_ref[0] = y[:, :128]
  o_ref[1] = y[:, 128:]


_tc_update = pl.pallas_call(
    _update_body,
    grid=(NRB,),
    in_specs=[
        pl.BlockSpec((NC, RB, 128), lambda i: (0, i, 0)),
        pl.BlockSpec((NC, RB, 128), lambda i: (0, i, 0)),
        pl.BlockSpec((1, H), lambda i: (0, 0)),
        pl.BlockSpec((H, H), lambda i: (0, 0)),
    ],
    out_specs=pl.BlockSpec((NC, RB, 128), lambda i: (0, i, 0)),
    out_shape=jax.ShapeDtypeStruct((NC, NP, 128), jnp.float32),
)


def _pool_head_body(z_ref, a_ref, b_ref, batch_ref, w1_ref, b1_ref,
                    w2_ref, b2_ref, o_ref, acc_ref):
  i = pl.program_id(0)

  @pl.when(i == 0)
  def _():
    acc_ref[...] = jnp.zeros_like(acc_ref)

  u0 = z_ref[0] + a_ref[0] + b_ref[:, :128]
  u1 = z_ref[1] + a_ref[1] + b_ref[:, 128:]
  ss = (jnp.sum(u0 * u0, axis=1, keepdims=True)
        + jnp.sum(u1 * u1, axis=1, keepdims=True))
  d = jnp.maximum(jnp.sqrt(ss), 1e-12)
  t0 = jnp.maximum(u0 / d, 0.0)
  t1 = jnp.maximum(u1 / d, 0.0)

  ids = batch_ref[0, 0, :]
  gi = lax.broadcasted_iota(jnp.int32, (G, RB), 0)
  onehot = (gi == ids[None, :]).astype(jnp.float32)
  acc_ref[:, :128] += jnp.dot(onehot, t0, preferred_element_type=jnp.float32)
  acc_ref[:, 128:] += jnp.dot(onehot, t1, preferred_element_type=jnp.float32)

  @pl.when(i == NRB - 1)
  def _():
    z = jnp.dot(acc_ref[...], w1_ref[...],
                preferred_element_type=jnp.float32) + b1_ref[...]
    z = jnp.maximum(z, 0.0)
    o_ref[...] = jnp.dot(z, w2_ref[...],
                         preferred_element_type=jnp.float32) + b2_ref[...]


_tc_pool_head = pl.pallas_call(
    _pool_head_body,
    grid=(NRB,),
    in_specs=[
        pl.BlockSpec((NC, RB, 128), lambda i: (0, i, 0)),
        pl.BlockSpec((NC, RB, 128), lambda i: (0, i, 0)),
        pl.BlockSpec((1, H), lambda i: (0, 0)),
        pl.BlockSpec((1, 1, RB), lambda i: (i, 0, 0)),
        pl.BlockSpec((H, H), lambda i: (0, 0)),
        pl.BlockSpec((1, H), lambda i: (0, 0)),
        pl.BlockSpec((H, 128), lambda i: (0, 0)),
        pl.BlockSpec((1, 128), lambda i: (0, 0)),
    ],
    out_specs=pl.BlockSpec((G, 128), lambda i: (0, 0)),
    out_shape=jax.ShapeDtypeStruct((G, 128), jnp.float32),
    scratch_shapes=[pltpu.VMEM((G, H), jnp.float32)],
)


def kernel(x, edge_index, batch, W0, b0, W1, b1, W2, b2,
           lin1_w, lin1_b, lin2_w, lin2_b):
  src = edge_index[0]
  dst = edge_index[1]
  srcr = src.reshape(NCH, K)
  dstr = dst.reshape(NCH, K)
  pk = jnp.stack([
      jnp.stack([srcr, dstr], axis=1),
      jnp.stack([srcr + NP, dstr], axis=1),
  ])
  zeros = jnp.zeros((SR, 128), jnp.float32)
  sc_agg = _make_sc_agg()

  xpad = jnp.pad(x, ((0, NP - N), (0, 0)))
  z1 = _tc_matmul0(xpad, W0)
  a1 = sc_agg(z1.reshape(2 * NP, 128), pk, zeros)
  z2 = _tc_update(z1, a1, b0.reshape(1, H), W1)
  a2 = sc_agg(z2.reshape(2 * NP, 128), pk, zeros)
  z3 = _tc_update(z2, a2, b1.reshape(1, H), W2)
  a3 = sc_agg(z3.reshape(2 * NP, 128), pk, zeros)

  batch_pad = jnp.pad(batch, (0, NP - N), constant_values=G)
  logits_pad = _tc_pool_head(
      z3, a3, b2.reshape(1, H), batch_pad.reshape(NRB, 1, RB),
      lin1_w, lin1_b.reshape(1, H),
      jnp.pad(lin2_w, ((0, 0), (0, 128 - C))),
      jnp.pad(lin2_b, (0, 128 - C)).reshape(1, 128),
  )
  return logits_pad[:, :C]
